# baseline (speedup 1.0000x reference)
import functools

import jax
import jax.numpy as jnp
from jax import lax
from jax.experimental import pallas as pl
from jax.experimental.pallas import tpu as pltpu
from jax.experimental.pallas import tpu_sc as plsc

B, L, D, H = 1024, 200, 64, 8
N = B * L
NC, NS = 2, 16
NW = NC * NS
IPS = 128
ROWS_PER_W = N // (NW * IPS)
KCH = 10
NCHUNK = ROWS_PER_W // KCH


def _sc_gather(idx1d, table):
    mesh = plsc.VectorSubcoreMesh(
        core_axis_name="c", subcore_axis_name="s", num_cores=NC, num_subcores=NS
    )

    @functools.partial(
        pl.kernel,
        out_type=jax.ShapeDtypeStruct((N // IPS, IPS, D), jnp.float32),
        mesh=mesh,
        scratch_types=[
            pltpu.VMEM((KCH * IPS,), jnp.int32),
            pltpu.VMEM((KCH, IPS, D), jnp.float32),
            pltpu.SemaphoreType.DMA,
        ],
        compiler_params=pltpu.CompilerParams(use_tc_tiling_on_sc=False),
    )
    def k(idx_hbm, table_hbm, out_hbm, idx_v, rows_v, sem):
        wid = lax.axis_index("s") * NC + lax.axis_index("c")
        base = wid * ROWS_PER_W

        def chunk(c, carry):
            off = base + c * KCH
            pltpu.sync_copy(idx_hbm.at[pl.ds(off * IPS, KCH * IPS)], idx_v)
            copies = [
                pltpu.async_copy(
                    table_hbm.at[idx_v.at[pl.ds(j * IPS, IPS)]], rows_v.at[j], sem
                )
                for j in range(KCH)
            ]
            for cp in copies:
                cp.wait()
            pltpu.sync_copy(rows_v, out_hbm.at[pl.ds(off, KCH)])
            return carry

        lax.fori_loop(0, NCHUNK, chunk, 0, unroll=False)

    return k(idx1d, table)


BB = 32


def _tc_body(v_ref, vid_ref, cm_ref, w1_ref, b1_ref, w2_ref, g_ref, out_ref, pm_ref):
    x = v_ref[...]
    t = jnp.tanh(x * w1_ref[...] + b1_ref[...])
    ve = lax.dot_general(
        t, w2_ref[...], (((1,), (0,)), ((), ())),
        preferred_element_type=jnp.float32,
    )
    ve = ve * cm_ref[...]
    out_ref[...] = ve + g_ref[...]
    pm_ref[...] = jnp.clip(vid_ref[...].astype(jnp.float32), 0.0, 1.0)


def _tc_combine(value_f, var_id, cm_f, W1, b1, W2, gathered_f):
    grid = (B // BB,)
    out_sum, pm = pl.pallas_call(
        _tc_body,
        grid=grid,
        in_specs=[
            pl.BlockSpec((BB * L, 1), lambda i: (i, 0)),
            pl.BlockSpec((BB, L), lambda i: (i, 0)),
            pl.BlockSpec((BB * L, 1), lambda i: (i, 0)),
            pl.BlockSpec((1, H), lambda i: (0, 0)),
            pl.BlockSpec((1, H), lambda i: (0, 0)),
            pl.BlockSpec((H, D), lambda i: (0, 0)),
            pl.BlockSpec((BB * L, D), lambda i: (i, 0)),
        ],
        out_specs=[
            pl.BlockSpec((BB * L, D), lambda i: (i, 0)),
            pl.BlockSpec((BB, L), lambda i: (i, 0)),
        ],
        out_shape=[
            jax.ShapeDtypeStruct((N, D), jnp.float32),
            jax.ShapeDtypeStruct((B, L), jnp.float32),
        ],
    )(value_f, var_id, cm_f, W1, b1, W2, gathered_f)
    return out_sum, pm


def kernel(value, var_id, category_mask, W1, b1, W2, emb_table):
    var_id = var_id.astype(jnp.int32)
    gathered = _sc_gather(var_id.reshape(N), emb_table)
    value_f = value.astype(jnp.float32).reshape(N, 1)
    cm_f = category_mask.astype(jnp.float32).reshape(N, 1)
    out_sum, pm = _tc_combine(
        value_f, var_id, cm_f,
        W1.reshape(1, H), b1.reshape(1, H), W2,
        gathered.reshape(N, D),
    )
    return (out_sum.reshape(B, L, D), pm)
